# baseline (speedup 1.0000x reference)
import jax
import jax.numpy as jnp
from jax.experimental import pallas as pl
from jax.experimental.pallas import tpu as pltpu

_SD_PAD = 8


def _mlp_body(m1t_ref, aux_ref, x_ref, o_ref, *, hidden_e, group):
    ht = jax.lax.dot_general(
        m1t_ref[...], x_ref[...],
        (((1,), (1,)), ((), ())),
        preferred_element_type=jnp.float32,
    )
    g = jnp.maximum(ht + aux_ref[:, 0:1], 0.0) * aux_ref[:, 1:2]
    s = g.reshape(hidden_e, group, x_ref.shape[0]).sum(axis=0)
    o_ref[...] = s


def kernel(x, params):
    B, sd = x.shape
    p_rows, hidden = params.shape
    assert p_rows == hidden + 1

    group = 128 // sd
    nrows = B // group
    assert B % group == 0

    w1 = params[:hidden, :sd]
    b1 = params[:hidden, sd]
    b2 = params[0, _SD_PAD]
    w2 = params[hidden, :hidden]

    hidden_e = hidden + 1
    w1e = jnp.concatenate([w1, jnp.zeros((1, sd), jnp.float32)], axis=0)
    b1e = jnp.concatenate([b1, jnp.ones((1,), jnp.float32)])
    w2e = jnp.concatenate([w2, b2[None]])

    eye = jnp.eye(group, dtype=jnp.float32)
    m1t = (w1e[:, None, None, :] * eye[None, :, :, None]).reshape(
        hidden_e * group, group * sd)
    aux = jnp.stack(
        [jnp.broadcast_to(b1e[:, None], (hidden_e, group)).reshape(-1),
         jnp.broadcast_to(w2e[:, None], (hidden_e, group)).reshape(-1)],
        axis=1)

    xr = x.reshape(nrows, group * sd)

    r_tile = 2048
    while nrows % r_tile:
        r_tile //= 2
    grid = (nrows // r_tile,)

    out = pl.pallas_call(
        lambda m, a, xx, o: _mlp_body(m, a, xx, o, hidden_e=hidden_e, group=group),
        grid=grid,
        in_specs=[
            pl.BlockSpec((hidden_e * group, group * sd), lambda i: (0, 0)),
            pl.BlockSpec((hidden_e * group, 2), lambda i: (0, 0)),
            pl.BlockSpec((r_tile, group * sd), lambda i: (i, 0)),
        ],
        out_specs=pl.BlockSpec((group, r_tile), lambda i: (0, i)),
        out_shape=jax.ShapeDtypeStruct((group, nrows), jnp.float32),
        compiler_params=pltpu.CompilerParams(
            dimension_semantics=("parallel",),
        ),
    )(m1t, aux, xr)

    return out.T.reshape(B, 1)
